# Initial kernel scaffold; baseline (speedup 1.0000x reference)
import jax
import jax.numpy as jnp
from jax.experimental import pallas as pl


def kernel(x, W1, b1, g1, be1, W2, b2, codebook):
    raise NotImplementedError("write your pallas kernel here")


# speedup vs baseline: 1.0836x; 1.0836x over previous
import functools

import jax
import jax.numpy as jnp
from jax import lax
from jax.experimental import pallas as pl
from jax.experimental.pallas import tpu as pltpu
from jax.experimental.pallas import tpu_sc as plsc

_B, _N, _IN_DIM, _HID, _CODE_DIM, _N_CODES = 32, 256, 768, 512, 256, 8192
_ROWS = _B * _N
_M = 512
_JB = 2048
_R = _ROWS // _M
_J = _N_CODES // _JB

_NC, _NS = 2, 16
_NW = _NC * _NS
_BPW = _ROWS // _NW


def _tc_body(x_ref, w1_ref, b1_ref, g1_ref, be1_ref, w2_ref, b2_ref, cb_ref,
             z_ref, tok_ref, zs_ref, best_ref, besti_ref):
    j = pl.program_id(1)

    @pl.when(j == 0)
    def _encode():
        h = jnp.dot(x_ref[...], w1_ref[...], preferred_element_type=jnp.float32)
        h = jnp.maximum(h + b1_ref[...], 0.0)
        mu = jnp.mean(h, axis=-1, keepdims=True)
        var = jnp.mean((h - mu) * (h - mu), axis=-1, keepdims=True)
        h = (h - mu) / jnp.sqrt(var + 1e-5) * g1_ref[...] + be1_ref[...]
        z = jnp.dot(h, w2_ref[...], preferred_element_type=jnp.float32)
        z = z + b2_ref[...]
        zs_ref[...] = z
        z_ref[...] = z
        best_ref[...] = jnp.full((1, _M), jnp.inf, jnp.float32)
        besti_ref[...] = jnp.zeros((1, _M), jnp.int32)

    z = zs_ref[...]
    cb = cb_ref[...]
    cnorm = jnp.sum(cb * cb, axis=1, keepdims=True)
    s = cnorm - 2.0 * lax.dot_general(
        cb, z, (((1,), (1,)), ((), ())), preferred_element_type=jnp.float32)
    lmin = jnp.min(s, axis=0, keepdims=True)
    row = lax.broadcasted_iota(jnp.int32, s.shape, 0)
    larg = jnp.min(jnp.where(s == lmin, row, _N_CODES),
                   axis=0, keepdims=True) + j * _JB
    better = lmin < best_ref[...]
    besti_ref[...] = jnp.where(better, larg, besti_ref[...])
    best_ref[...] = jnp.where(better, lmin, best_ref[...])

    @pl.when(j == _J - 1)
    def _emit_tokens():
        tok_ref[...] = besti_ref[...].reshape(1, 1, _M)


def _encode_quantize(x2d, w1, b1, g1, be1, w2, b2, codebook):
    grid = (_R, _J)
    z, tok = pl.pallas_call(
        _tc_body,
        grid=grid,
        in_specs=[
            pl.BlockSpec((_M, _IN_DIM), lambda r, j: (r, 0)),
            pl.BlockSpec((_IN_DIM, _HID), lambda r, j: (0, 0)),
            pl.BlockSpec((1, _HID), lambda r, j: (0, 0)),
            pl.BlockSpec((1, _HID), lambda r, j: (0, 0)),
            pl.BlockSpec((1, _HID), lambda r, j: (0, 0)),
            pl.BlockSpec((_HID, _CODE_DIM), lambda r, j: (0, 0)),
            pl.BlockSpec((1, _CODE_DIM), lambda r, j: (0, 0)),
            pl.BlockSpec((_JB, _CODE_DIM), lambda r, j: (j, 0)),
        ],
        out_specs=[
            pl.BlockSpec((_M, _CODE_DIM), lambda r, j: (r, 0)),
            pl.BlockSpec((1, 1, _M), lambda r, j: (r, 0, 0)),
        ],
        out_shape=[
            jax.ShapeDtypeStruct((_ROWS, _CODE_DIM), jnp.float32),
            jax.ShapeDtypeStruct((_R, 1, _M), jnp.int32),
        ],
        scratch_shapes=[
            pltpu.VMEM((_M, _CODE_DIM), jnp.float32),
            pltpu.VMEM((1, _M), jnp.float32),
            pltpu.VMEM((1, _M), jnp.int32),
        ],
        compiler_params=pltpu.CompilerParams(
            dimension_semantics=("arbitrary", "arbitrary")),
    )(x2d, w1, b1, g1, be1, w2, b2, codebook)
    return z, tok.reshape(_ROWS)


@functools.cache
def _make_sc_gather():
    mesh = plsc.VectorSubcoreMesh(core_axis_name="c", subcore_axis_name="s")

    @functools.partial(
        pl.kernel,
        mesh=mesh,
        out_type=jax.ShapeDtypeStruct((_ROWS, _CODE_DIM), jnp.float32),
        scratch_types=[
            pltpu.VMEM((_BPW,), jnp.int32),
            pltpu.VMEM((_BPW, _CODE_DIM), jnp.float32),
            pltpu.SemaphoreType.DMA,
        ],
    )
    def _sc_gather(cb_hbm, idx_hbm, out_hbm, idx_v, rows_v, sem):
        wid = lax.axis_index("s") * _NC + lax.axis_index("c")
        base = wid * _BPW
        pltpu.sync_copy(idx_hbm.at[pl.ds(base, _BPW)], idx_v)
        pltpu.async_copy(cb_hbm.at[idx_v], rows_v, sem).wait()
        pltpu.sync_copy(rows_v, out_hbm.at[pl.ds(base, _BPW)])

    return _sc_gather


def kernel(x, W1, b1, g1, be1, W2, b2, codebook):
    x2d = x.reshape(_ROWS, _IN_DIM)
    z_flat, tokens = _encode_quantize(
        x2d, W1, b1.reshape(1, _HID), g1.reshape(1, _HID),
        be1.reshape(1, _HID), W2, b2.reshape(1, _CODE_DIM), codebook)
    z_q = _make_sc_gather()(codebook, tokens)
    emb = z_flat + (z_q - z_flat)
    return (tokens.reshape(_B, _N),
            emb.reshape(_B, _N, _CODE_DIM),
            z_flat.reshape(_B, _N, _CODE_DIM))
